# baseline (speedup 1.0000x reference)
import jax
import jax.numpy as jnp
from jax import lax
from jax.experimental import pallas as pl
from jax.experimental.pallas import tpu as pltpu
from jax.experimental.pallas import tpu_sc as plsc

NUM_EXPERTS = 8
TOP_K = 2
HIDDEN = 1024
BT = 2048
NC, NS, L = 2, 16, 16
NW = NC * NS
TPW = 1024
GROUPS = TPW // L


def _logits_block(x_ref, w_ref, logits_ref):
    x = x_ref[...]
    w = w_ref[...]
    logits_ref[0] = jax.lax.dot_general(
        w, x,
        dimension_numbers=(((1,), (1,)), ((), ())),
        preferred_element_type=jnp.float32,
    )


def _tc_logits(x, W):
    T = x.shape[0]
    nblk = T // BT
    return pl.pallas_call(
        _logits_block,
        grid=(nblk,),
        in_specs=[
            pl.BlockSpec((BT, HIDDEN), lambda i: (i, 0)),
            pl.BlockSpec((NUM_EXPERTS, HIDDEN), lambda i: (0, 0)),
        ],
        out_specs=pl.BlockSpec((1, NUM_EXPERTS, BT), lambda i: (i, 0, 0)),
        out_shape=jax.ShapeDtypeStruct((nblk, NUM_EXPERTS, BT), jnp.float32),
    )(x, W)


def _sc_route_body(logits_hbm, logits_out, aff_out, idx_out,
                   logits_v, logits_tm, aff_tm, idx_tm):
    halves = BT // TPW
    wid = lax.axis_index("s") * NC + lax.axis_index("c")
    n = wid // halves
    h = wid % halves
    base = h * TPW

    pltpu.sync_copy(logits_hbm.at[n, :, pl.ds(base, TPW)], logits_v)

    tok0 = lax.iota(jnp.int32, L)

    def gbody(g, _):
        off = g * L
        toks = tok0 + off
        l = [logits_v[e, pl.ds(off, L)] for e in range(NUM_EXPERTS)]
        m = l[0]
        for e in range(1, NUM_EXPERTS):
            m = jnp.maximum(m, l[e])
        ex = [jnp.exp(v - m) for v in l]
        s = ex[0]
        for e in range(1, NUM_EXPERTS):
            s = s + ex[e]
        r = 1.0 / s
        a = [v * r for v in ex]
        pos0 = toks * NUM_EXPERTS
        for e in range(NUM_EXPERTS):
            pos = pos0 + e
            plsc.store_scatter(logits_tm, [pos], l[e])
            plsc.store_scatter(aff_tm, [pos], a[e])
        best = a[0]
        bidx = jnp.zeros((L,), jnp.int32)
        second = jnp.full((L,), -1.0, jnp.float32)
        sidx = jnp.zeros((L,), jnp.int32)
        for e in range(1, NUM_EXPERTS):
            esp = jnp.full((L,), e, jnp.int32)
            gt_best = a[e] > best
            gt_sec = a[e] > second
            second = jnp.where(gt_best, best, jnp.where(gt_sec, a[e], second))
            sidx = jnp.where(gt_best, bidx, jnp.where(gt_sec, esp, sidx))
            best = jnp.where(gt_best, a[e], best)
            bidx = jnp.where(gt_best, esp, bidx)
        ipos = toks * TOP_K
        plsc.store_scatter(idx_tm, [ipos], bidx)
        plsc.store_scatter(idx_tm, [ipos + 1], sidx)
        return 0

    lax.fori_loop(0, GROUPS, gbody, 0)

    row0 = n * BT + base
    pltpu.sync_copy(logits_tm, logits_out.at[pl.ds(row0 * NUM_EXPERTS, TPW * NUM_EXPERTS)])
    pltpu.sync_copy(aff_tm, aff_out.at[pl.ds(row0 * NUM_EXPERTS, TPW * NUM_EXPERTS)])
    pltpu.sync_copy(idx_tm, idx_out.at[pl.ds(row0 * TOP_K, TPW * TOP_K)])


def _sc_route(logits_t, T):
    mesh = plsc.VectorSubcoreMesh(core_axis_name="c", subcore_axis_name="s")
    k = pl.kernel(
        _sc_route_body,
        out_type=[
            jax.ShapeDtypeStruct((T * NUM_EXPERTS,), jnp.float32),
            jax.ShapeDtypeStruct((T * NUM_EXPERTS,), jnp.float32),
            jax.ShapeDtypeStruct((T * TOP_K,), jnp.int32),
        ],
        mesh=mesh,
        compiler_params=pltpu.CompilerParams(needs_layout_passes=False),
        scratch_types=[
            pltpu.VMEM((NUM_EXPERTS, TPW), jnp.float32),
            pltpu.VMEM((TPW * NUM_EXPERTS,), jnp.float32),
            pltpu.VMEM((TPW * NUM_EXPERTS,), jnp.float32),
            pltpu.VMEM((TPW * TOP_K,), jnp.int32),
        ],
    )
    return k(logits_t)


@jax.jit
def _router(x, W):
    T = x.shape[0]
    logits_t = _tc_logits(x, W)
    logits_f, aff_f, idx_f = _sc_route(logits_t, T)
    return (logits_f.reshape(T, NUM_EXPERTS),
            aff_f.reshape(T, NUM_EXPERTS),
            idx_f.reshape(T, TOP_K))


def kernel(hidden_states, W):
    B, S, H = hidden_states.shape
    x = hidden_states.reshape(B * S, H)
    return _router(x, W)


# speedup vs baseline: 2.1103x; 2.1103x over previous
import jax
import jax.numpy as jnp
from jax import lax
from jax.experimental import pallas as pl
from jax.experimental.pallas import tpu as pltpu
from jax.experimental.pallas import tpu_sc as plsc

NUM_EXPERTS = 8
TOP_K = 2
HIDDEN = 1024
BT = 2048
NC, NS, L = 2, 16, 16
NW = NC * NS
TPW = 32768 // NW
GROUPS = TPW // L


def _logits_block(x_ref, w_ref, logits_ref):
    x = x_ref[...]
    w = w_ref[...]
    logits_ref[...] = jax.lax.dot_general(
        w, x,
        dimension_numbers=(((1,), (1,)), ((), ())),
        preferred_element_type=jnp.float32,
    )


def _tc_logits(x, W):
    T = x.shape[0]
    nblk = T // BT
    return pl.pallas_call(
        _logits_block,
        grid=(nblk,),
        in_specs=[
            pl.BlockSpec((BT, HIDDEN), lambda i: (i, 0)),
            pl.BlockSpec((NUM_EXPERTS, HIDDEN), lambda i: (0, 0)),
        ],
        out_specs=pl.BlockSpec((NUM_EXPERTS, BT), lambda i: (0, i)),
        out_shape=jax.ShapeDtypeStruct((NUM_EXPERTS, T), jnp.float32),
    )(x, W)


def _sc_route_body(logits_hbm, aff_out, idx_out, logits_v, aff_v, idx_v):
    wid = lax.axis_index("s") * NC + lax.axis_index("c")
    tok0 = wid * TPW

    pltpu.sync_copy(logits_hbm.at[:, pl.ds(tok0, TPW)], logits_v)

    def gbody(g, _):
        off = g * L
        l = [logits_v[e, pl.ds(off, L)] for e in range(NUM_EXPERTS)]
        m = l[0]
        for e in range(1, NUM_EXPERTS):
            m = jnp.maximum(m, l[e])
        ex = [jnp.exp(v - m) for v in l]
        s = ex[0]
        for e in range(1, NUM_EXPERTS):
            s = s + ex[e]
        r = 1.0 / s
        a = [v * r for v in ex]
        for e in range(NUM_EXPERTS):
            aff_v[e, pl.ds(off, L)] = a[e]
        best = a[0]
        bidx = jnp.zeros((L,), jnp.int32)
        second = jnp.full((L,), -1.0, jnp.float32)
        sidx = jnp.zeros((L,), jnp.int32)
        for e in range(1, NUM_EXPERTS):
            esp = jnp.full((L,), e, jnp.int32)
            gt_best = a[e] > best
            gt_sec = a[e] > second
            second = jnp.where(gt_best, best, jnp.where(gt_sec, a[e], second))
            sidx = jnp.where(gt_best, bidx, jnp.where(gt_sec, esp, sidx))
            best = jnp.where(gt_best, a[e], best)
            bidx = jnp.where(gt_best, esp, bidx)
        idx_v[0, pl.ds(off, L)] = bidx
        idx_v[1, pl.ds(off, L)] = sidx
        return 0

    lax.fori_loop(0, GROUPS, gbody, 0)

    pltpu.sync_copy(aff_v, aff_out.at[:, pl.ds(tok0, TPW)])
    pltpu.sync_copy(idx_v, idx_out.at[:, pl.ds(tok0, TPW)])


def _sc_route(logits_w, T):
    mesh = plsc.VectorSubcoreMesh(core_axis_name="c", subcore_axis_name="s")
    k = pl.kernel(
        _sc_route_body,
        out_type=[
            jax.ShapeDtypeStruct((NUM_EXPERTS, T), jnp.float32),
            jax.ShapeDtypeStruct((TOP_K, T), jnp.int32),
        ],
        mesh=mesh,
        compiler_params=pltpu.CompilerParams(skip_device_barrier=True),
        scratch_types=[
            pltpu.VMEM((NUM_EXPERTS, TPW), jnp.float32),
            pltpu.VMEM((NUM_EXPERTS, TPW), jnp.float32),
            pltpu.VMEM((TOP_K, TPW), jnp.int32),
        ],
    )
    return k(logits_w)


@jax.jit
def _router(x, W):
    T = x.shape[0]
    logits_w = _tc_logits(x, W)
    aff_w, idx_w = _sc_route(logits_w, T)
    return logits_w.T, aff_w.T, idx_w.T


def kernel(hidden_states, W):
    B, S, H = hidden_states.shape
    x = hidden_states.reshape(B * S, H)
    return _router(x, W)


# speedup vs baseline: 3.1227x; 1.4797x over previous
import jax
import jax.numpy as jnp
from jax.experimental import pallas as pl

NUM_EXPERTS = 8
TOP_K = 2
HIDDEN = 1024
BT = 2048


def _router_block(x_ref, w_ref, logits_ref, aff_ref, idx_ref):
    x = x_ref[...]
    w = w_ref[...]
    logits = jax.lax.dot_general(
        w, x,
        dimension_numbers=(((1,), (1,)), ((), ())),
        preferred_element_type=jnp.float32,
    )
    m = jnp.max(logits, axis=0, keepdims=True)
    e = jnp.exp(logits - m)
    s = jnp.sum(e, axis=0, keepdims=True)
    aff = e * (1.0 / s)

    iota = jax.lax.broadcasted_iota(jnp.int32, aff.shape, 0)
    big = jnp.int32(NUM_EXPERTS)
    v1 = jnp.max(aff, axis=0, keepdims=True)
    idx1 = jnp.min(jnp.where(aff == v1, iota, big), axis=0, keepdims=True)
    aff2 = jnp.where(iota == idx1, -1.0, aff)
    v2 = jnp.max(aff2, axis=0, keepdims=True)
    idx2 = jnp.min(jnp.where(aff2 == v2, iota, big), axis=0, keepdims=True)

    logits_ref[...] = logits
    aff_ref[...] = aff
    idx_ref[...] = jnp.concatenate([idx1, idx2], axis=0)


@jax.jit
def _router(x, W):
    T = x.shape[0]
    nblk = T // BT
    logits_w, aff_w, idx_w = pl.pallas_call(
        _router_block,
        grid=(nblk,),
        in_specs=[
            pl.BlockSpec((BT, HIDDEN), lambda i: (i, 0)),
            pl.BlockSpec((NUM_EXPERTS, HIDDEN), lambda i: (0, 0)),
        ],
        out_specs=[
            pl.BlockSpec((NUM_EXPERTS, BT), lambda i: (0, i)),
            pl.BlockSpec((NUM_EXPERTS, BT), lambda i: (0, i)),
            pl.BlockSpec((TOP_K, BT), lambda i: (0, i)),
        ],
        out_shape=[
            jax.ShapeDtypeStruct((NUM_EXPERTS, T), jnp.float32),
            jax.ShapeDtypeStruct((NUM_EXPERTS, T), jnp.float32),
            jax.ShapeDtypeStruct((TOP_K, T), jnp.int32),
        ],
    )(x, W)
    return logits_w.T, aff_w.T, idx_w.T


def kernel(hidden_states, W):
    B, S, H = hidden_states.shape
    x = hidden_states.reshape(B * S, H)
    return _router(x, W)
